# baseline (speedup 1.0000x reference)
import functools

import jax
import jax.numpy as jnp
from jax import lax
from jax.experimental import pallas as pl
from jax.experimental.pallas import tpu as pltpu
from jax.experimental.pallas import tpu_sc as plsc

_F = 100000
_NC, _NS, _L = 2, 16, 16
_CH = 3200
_H1 = 1664
_H2 = _CH - _H1
_LAST = 96896


def _sc_body(int_ref, out_ref, xb, ov, sem0, sem1):
    cid = lax.axis_index("c")
    sid = lax.axis_index("s")
    wid = sid * _NC + cid
    off = pl.multiple_of(jnp.minimum(wid * _CH, _LAST), 128)

    c0 = pltpu.async_copy(
        int_ref.at[:, pl.ds(off, _H1)], xb.at[:, pl.ds(0, _H1)], sem0
    )
    c1 = pltpu.async_copy(
        int_ref.at[:, pl.ds(off + _H1, _H2)], xb.at[:, pl.ds(_H1, _H2)], sem1
    )

    def step(i, carry):
        sl = pl.ds(i * _L, _L)
        ov[0, sl] = xb[0, sl] + xb[1, sl] + xb[2, sl]
        return carry

    c0.wait()
    lax.fori_loop(0, _H1 // _L, step, 0)
    c1.wait()
    lax.fori_loop(_H1 // _L, _CH // _L, step, 0)
    pltpu.sync_copy(ov, out_ref.at[:, pl.ds(off, _CH)])


@functools.partial(
    pl.kernel,
    mesh=plsc.VectorSubcoreMesh(core_axis_name="c", subcore_axis_name="s"),
    out_type=jax.ShapeDtypeStruct((1, _F), jnp.float32),
    scratch_types=[
        pltpu.VMEM((3, _CH), jnp.float32),
        pltpu.VMEM((1, _CH), jnp.float32),
        pltpu.SemaphoreType.DMA,
        pltpu.SemaphoreType.DMA,
    ],
)
def _sc_pool(int_ref, out_ref, xb, ov, sem0, sem1):
    _sc_body(int_ref, out_ref, xb, ov, sem0, sem1)


def kernel(intensities, avg_dist, W_gcn, b_gcn, W_attn, b_attn):
    return _sc_pool(intensities)
